# Initial kernel scaffold; baseline (speedup 1.0000x reference)
import jax
import jax.numpy as jnp
from jax.experimental import pallas as pl


def kernel(inp, edge_index, weights, bias):
    raise NotImplementedError("write your pallas kernel here")


# speedup vs baseline: 2.2751x; 2.2751x over previous
import functools

import jax
import jax.numpy as jnp
from jax import lax
from jax.experimental import pallas as pl
from jax.experimental.pallas import tpu as pltpu
from jax.experimental.pallas import tpu_sc as plsc

N = 10000
E = 320000
R = 4
D = 128

NC = 2
NS = 16
NW = NC * NS

CHUNK = 128
EP = 1310720
EDGES_PER_TILE = EP // NW
CHUNKS_PER_TILE = EDGES_PER_TILE // CHUNK
NACC = 10112
ROWS_PER_TILE = NACC // NS


def _mm_body(x_ref, w_ref, o_ref):
    o_ref[0] = jnp.dot(x_ref[...], w_ref[0], preferred_element_type=jnp.float32)


def _relation_matmuls(inp, weights):
    BN = 1000
    return pl.pallas_call(
        _mm_body,
        grid=(R, N // BN),
        in_specs=[
            pl.BlockSpec((BN, D), lambda r, i: (i, 0)),
            pl.BlockSpec((1, D, D), lambda r, i: (r, 0, 0)),
        ],
        out_specs=pl.BlockSpec((1, BN, D), lambda r, i: (r, i, 0)),
        out_shape=jax.ShapeDtypeStruct((R, N, D), jnp.float32),
    )(inp, weights)


def _sc_body(table_hbm, src_hbm, dst_hbm, out0_hbm, out1_hbm,
             src_v, dst_v, rows_v, acc_sh, sem):
    c = lax.axis_index("c")
    s = lax.axis_index("s")
    wid = s * NC + c

    def zrow(i, _):
        def zcol(j, _):
            rows_v[i, pl.ds(j * 16, 16)] = jnp.zeros((16,), jnp.float32)
            return 0
        return lax.fori_loop(0, D // 16, zcol, 0)
    lax.fori_loop(0, CHUNK, zrow, 0)

    row0 = s * ROWS_PER_TILE
    for k in range(ROWS_PER_TILE // CHUNK):
        pltpu.sync_copy(rows_v, acc_sh.at[pl.ds(row0 + k * CHUNK, CHUNK)])
    tail = ROWS_PER_TILE % CHUNK
    pltpu.sync_copy(rows_v.at[pl.ds(0, tail)],
                    acc_sh.at[pl.ds(row0 + (ROWS_PER_TILE // CHUNK) * CHUNK, tail)])

    plsc.subcore_barrier()

    base = wid * EDGES_PER_TILE

    def step(j, _):
        off = base + j * CHUNK
        pltpu.sync_copy(src_hbm.at[pl.ds(off, CHUNK)], src_v)
        pltpu.async_copy(table_hbm.at[src_v], rows_v, sem).wait()
        pltpu.sync_copy(dst_hbm.at[pl.ds(off, CHUNK)], dst_v)
        pltpu.sync_copy(rows_v, acc_sh.at[dst_v], add=True)
        return 0
    lax.fori_loop(0, CHUNKS_PER_TILE, step, 0)

    plsc.subcore_barrier()

    @pl.when(c == 0)
    def _():
        pltpu.sync_copy(acc_sh.at[pl.ds(row0, ROWS_PER_TILE)],
                        out0_hbm.at[pl.ds(row0, ROWS_PER_TILE)])

    @pl.when(c == 1)
    def _():
        pltpu.sync_copy(acc_sh.at[pl.ds(row0, ROWS_PER_TILE)],
                        out1_hbm.at[pl.ds(row0, ROWS_PER_TILE)])


_sc_spmm = functools.partial(
    pl.kernel,
    out_type=(
        jax.ShapeDtypeStruct((NACC, D), jnp.float32),
        jax.ShapeDtypeStruct((NACC, D), jnp.float32),
    ),
    mesh=plsc.VectorSubcoreMesh(core_axis_name="c", subcore_axis_name="s"),
    scratch_types=[
        pltpu.VMEM((CHUNK,), jnp.int32),
        pltpu.VMEM((CHUNK,), jnp.int32),
        pltpu.VMEM((CHUNK, D), jnp.float32),
        pltpu.VMEM_SHARED((NACC, D), jnp.float32),
        pltpu.SemaphoreType.DMA,
    ],
)(_sc_body)


def _combine_body(p0_ref, p1_ref, b_ref, o_ref):
    bias_sum = jnp.sum(b_ref[...], axis=0, keepdims=True)
    o_ref[...] = p0_ref[...] + p1_ref[...] + bias_sum


def _combine(p0, p1, bias):
    BN = 400
    return pl.pallas_call(
        _combine_body,
        grid=(N // BN,),
        in_specs=[
            pl.BlockSpec((BN, D), lambda i: (i, 0)),
            pl.BlockSpec((BN, D), lambda i: (i, 0)),
            pl.BlockSpec((R, D), lambda i: (0, 0)),
        ],
        out_specs=pl.BlockSpec((BN, D), lambda i: (i, 0)),
        out_shape=jax.ShapeDtypeStruct((N, D), jnp.float32),
    )(p0, p1, bias)


@jax.jit
def kernel(inp, edge_index, weights, bias):
    table = _relation_matmuls(inp, weights).reshape(R * N, D)

    roff = (jnp.arange(R, dtype=jnp.int32) * N)[:, None]
    src = (edge_index[:, 1, :] + roff).reshape(-1)
    dst = edge_index[:, 0, :].reshape(-1)
    npad = EP - R * E
    src = jnp.concatenate([src, jnp.zeros((npad,), jnp.int32)])
    dst = jnp.concatenate([dst, jnp.full((npad,), N, jnp.int32)])

    p0, p1 = _sc_spmm(table, src, dst)
    return _combine(p0, p1, bias)


# speedup vs baseline: 3.2151x; 1.4131x over previous
import functools

import jax
import jax.numpy as jnp
from jax import lax
from jax.experimental import pallas as pl
from jax.experimental.pallas import tpu as pltpu
from jax.experimental.pallas import tpu_sc as plsc

N = 10000
E = 320000
R = 4
D = 128

NC = 2
NS = 16
NW = NC * NS

CHUNK = 128
EP = 1310720
EDGES_PER_TILE = EP // NW
CHUNKS_PER_TILE = EDGES_PER_TILE // CHUNK
NACC = 10112
ROWS_PER_TILE = NACC // NS


def _mm_body(x_ref, w_ref, o_ref):
    o_ref[0] = jnp.dot(x_ref[...], w_ref[0], preferred_element_type=jnp.float32)


def _relation_matmuls(inp, weights):
    BN = 1000
    return pl.pallas_call(
        _mm_body,
        grid=(R, N // BN),
        in_specs=[
            pl.BlockSpec((BN, D), lambda r, i: (i, 0)),
            pl.BlockSpec((1, D, D), lambda r, i: (r, 0, 0)),
        ],
        out_specs=pl.BlockSpec((1, BN, D), lambda r, i: (r, i, 0)),
        out_shape=jax.ShapeDtypeStruct((R, N, D), jnp.float32),
    )(inp, weights)


IB = 32
NB = CHUNKS_PER_TILE // IB


def _sc_body(table_hbm, src_hbm, dst_hbm, out0_hbm, out1_hbm,
             src_a, src_b, dst_a, dst_b, rows0, rows1, acc_sh,
             sem_g0, sem_g1, sem_s0, sem_s1, sem_i0, sem_i1):
    c = lax.axis_index("c")
    s = lax.axis_index("s")
    wid = s * NC + c

    def zrow(i, _):
        def zcol(j, _):
            rows0[i, pl.ds(j * 16, 16)] = jnp.zeros((16,), jnp.float32)
            return 0
        return lax.fori_loop(0, D // 16, zcol, 0)
    lax.fori_loop(0, CHUNK, zrow, 0)

    row0 = s * ROWS_PER_TILE
    for k in range(ROWS_PER_TILE // CHUNK):
        pltpu.sync_copy(rows0, acc_sh.at[pl.ds(row0 + k * CHUNK, CHUNK)])
    tail = ROWS_PER_TILE % CHUNK
    pltpu.sync_copy(rows0.at[pl.ds(0, tail)],
                    acc_sh.at[pl.ds(row0 + (ROWS_PER_TILE // CHUNK) * CHUNK, tail)])

    plsc.subcore_barrier()

    cbase = wid * CHUNKS_PER_TILE
    idx_bufs = [(src_a, dst_a, sem_i0), (src_b, dst_b, sem_i1)]

    def fetch_idx(p):
        sblk, dblk, sem = idx_bufs[p % 2]
        pltpu.async_copy(src_hbm.at[pl.ds(cbase + p * IB, IB)], sblk, sem)
        pltpu.async_copy(dst_hbm.at[pl.ds(cbase + p * IB, IB)], dblk, sem)

    def wait_idx(p):
        sblk, dblk, sem = idx_bufs[p % 2]
        pltpu.make_async_copy(src_hbm.at[pl.ds(0, IB)], sblk, sem).wait()
        pltpu.make_async_copy(dst_hbm.at[pl.ds(0, IB)], dblk, sem).wait()

    def wait_gather(buf, sem):
        pltpu.make_async_copy(table_hbm.at[pl.ds(0, CHUNK)], buf, sem).wait()

    fetch_idx(0)
    for p in range(NB):
        sblk, dblk, _ = idx_bufs[p % 2]
        if p + 1 < NB:
            fetch_idx(p + 1)
        wait_idx(p)

        pltpu.async_copy(table_hbm.at[sblk.at[0]], rows0, sem_g0)
        pltpu.async_copy(table_hbm.at[sblk.at[1]], rows1, sem_g1)

        def pair(g, _, sblk=sblk, dblk=dblk):
            j0 = 2 * g
            j1 = 2 * g + 1
            wait_gather(rows0, sem_g0)
            s0 = pltpu.async_copy(rows0, acc_sh.at[dblk.at[j0]], sem_s0, add=True)
            wait_gather(rows1, sem_g1)
            s1 = pltpu.async_copy(rows1, acc_sh.at[dblk.at[j1]], sem_s1, add=True)
            s0.wait()

            @pl.when(j0 + 2 < IB)
            def _():
                pltpu.async_copy(table_hbm.at[sblk.at[j0 + 2]], rows0, sem_g0)
            s1.wait()

            @pl.when(j1 + 2 < IB)
            def _():
                pltpu.async_copy(table_hbm.at[sblk.at[j1 + 2]], rows1, sem_g1)
            return 0
        lax.fori_loop(0, IB // 2, pair, 0)

    plsc.subcore_barrier()

    @pl.when(c == 0)
    def _():
        pltpu.sync_copy(acc_sh.at[pl.ds(row0, ROWS_PER_TILE)],
                        out0_hbm.at[pl.ds(row0, ROWS_PER_TILE)])

    @pl.when(c == 1)
    def _():
        pltpu.sync_copy(acc_sh.at[pl.ds(row0, ROWS_PER_TILE)],
                        out1_hbm.at[pl.ds(row0, ROWS_PER_TILE)])


_sc_spmm = functools.partial(
    pl.kernel,
    out_type=(
        jax.ShapeDtypeStruct((NACC, D), jnp.float32),
        jax.ShapeDtypeStruct((NACC, D), jnp.float32),
    ),
    mesh=plsc.VectorSubcoreMesh(core_axis_name="c", subcore_axis_name="s"),
    scratch_types=[
        pltpu.VMEM((IB, CHUNK), jnp.int32),
        pltpu.VMEM((IB, CHUNK), jnp.int32),
        pltpu.VMEM((IB, CHUNK), jnp.int32),
        pltpu.VMEM((IB, CHUNK), jnp.int32),
        pltpu.VMEM((CHUNK, D), jnp.float32),
        pltpu.VMEM((CHUNK, D), jnp.float32),
        pltpu.VMEM_SHARED((NACC, D), jnp.float32),
        pltpu.SemaphoreType.DMA,
        pltpu.SemaphoreType.DMA,
        pltpu.SemaphoreType.DMA,
        pltpu.SemaphoreType.DMA,
        pltpu.SemaphoreType.DMA,
        pltpu.SemaphoreType.DMA,
    ],
)(_sc_body)


def _combine_body(p0_ref, p1_ref, b_ref, o_ref):
    bias_sum = jnp.sum(b_ref[...], axis=0, keepdims=True)
    o_ref[...] = p0_ref[...] + p1_ref[...] + bias_sum


def _combine(p0, p1, bias):
    BN = 400
    return pl.pallas_call(
        _combine_body,
        grid=(N // BN,),
        in_specs=[
            pl.BlockSpec((BN, D), lambda i: (i, 0)),
            pl.BlockSpec((BN, D), lambda i: (i, 0)),
            pl.BlockSpec((R, D), lambda i: (0, 0)),
        ],
        out_specs=pl.BlockSpec((BN, D), lambda i: (i, 0)),
        out_shape=jax.ShapeDtypeStruct((N, D), jnp.float32),
    )(p0, p1, bias)


@jax.jit
def kernel(inp, edge_index, weights, bias):
    table = _relation_matmuls(inp, weights).reshape(R * N, D)

    roff = (jnp.arange(R, dtype=jnp.int32) * N)[:, None]
    src = (edge_index[:, 1, :] + roff).reshape(-1)
    dst = edge_index[:, 0, :].reshape(-1)
    npad = EP - R * E
    src = jnp.concatenate([src, jnp.zeros((npad,), jnp.int32)])
    dst = jnp.concatenate([dst, jnp.full((npad,), N, jnp.int32)])
    src = src.reshape(EP // CHUNK, CHUNK)
    dst = dst.reshape(EP // CHUNK, CHUNK)

    p0, p1 = _sc_spmm(table, src, dst)
    return _combine(p0, p1, bias)


# speedup vs baseline: 8.8372x; 2.7487x over previous
import functools

import jax
import jax.numpy as jnp
from jax import lax
from jax.experimental import pallas as pl
from jax.experimental.pallas import tpu as pltpu
from jax.experimental.pallas import tpu_sc as plsc

N = 10000
E = 320000
R = 4
D = 128

NC = 2
NS = 16
NW = NC * NS

CHUNK = 128
EP = 1310720
EDGES_PER_TILE = EP // NW
CHUNKS_PER_TILE = EDGES_PER_TILE // CHUNK
NACC = 10112
ROWS_PER_TILE = NACC // NS


def _mm_body(x_ref, w_ref, o_ref):
    o_ref[0] = jnp.dot(x_ref[...], w_ref[0], preferred_element_type=jnp.float32)


def _relation_matmuls(inp, weights):
    BN = 1000
    return pl.pallas_call(
        _mm_body,
        grid=(R, N // BN),
        in_specs=[
            pl.BlockSpec((BN, D), lambda r, i: (i, 0)),
            pl.BlockSpec((1, D, D), lambda r, i: (r, 0, 0)),
        ],
        out_specs=pl.BlockSpec((1, BN, D), lambda r, i: (r, i, 0)),
        out_shape=jax.ShapeDtypeStruct((R, N, D), jnp.float32),
    )(inp, weights)


IB = 32
NB = CHUNKS_PER_TILE // IB


def _sc_body(table_hbm, src_hbm, dst_hbm, out0_hbm, out1_hbm,
             src_a, src_b, dst_a, dst_b, rows0, rows1, acc_sh,
             sem_g0, sem_g1, sem_s0, sem_s1, sem_i0, sem_i1):
    c = lax.axis_index("c")
    s = lax.axis_index("s")
    wid = s * NC + c

    def zrow(i, _):
        def zcol(j, _):
            rows0[i, pl.ds(j * 16, 16)] = jnp.zeros((16,), jnp.float32)
            return 0
        return lax.fori_loop(0, D // 16, zcol, 0)
    lax.fori_loop(0, CHUNK, zrow, 0)

    row0 = s * ROWS_PER_TILE
    for k in range(ROWS_PER_TILE // CHUNK):
        pltpu.sync_copy(rows0, acc_sh.at[pl.ds(row0 + k * CHUNK, CHUNK)])
    tail = ROWS_PER_TILE % CHUNK
    pltpu.sync_copy(rows0.at[pl.ds(0, tail)],
                    acc_sh.at[pl.ds(row0 + (ROWS_PER_TILE // CHUNK) * CHUNK, tail)])

    plsc.subcore_barrier()

    cbase = wid * CHUNKS_PER_TILE
    idx_bufs = [(src_a, dst_a, sem_i0), (src_b, dst_b, sem_i1)]

    def fetch_idx(p):
        sblk, dblk, sem = idx_bufs[p % 2]
        pltpu.async_copy(src_hbm.at[pl.ds(cbase + p * IB, IB)], sblk, sem)
        pltpu.async_copy(dst_hbm.at[pl.ds(cbase + p * IB, IB)], dblk, sem)

    def wait_idx(p):
        sblk, dblk, sem = idx_bufs[p % 2]
        pltpu.make_async_copy(src_hbm.at[pl.ds(0, IB)], sblk, sem).wait()
        pltpu.make_async_copy(dst_hbm.at[pl.ds(0, IB)], dblk, sem).wait()

    def wait_gather(buf, sem):
        pltpu.make_async_copy(table_hbm.at[pl.ds(0, CHUNK)], buf, sem).wait()

    fetch_idx(0)
    for p in range(NB):
        sblk, dblk, _ = idx_bufs[p % 2]
        if p + 1 < NB:
            fetch_idx(p + 1)
        wait_idx(p)

        pltpu.async_copy(table_hbm.at[sblk.at[0]], rows0, sem_g0)
        pltpu.async_copy(table_hbm.at[sblk.at[1]], rows1, sem_g1)

        def pair(g, _, sblk=sblk, dblk=dblk):
            j0 = 2 * g
            j1 = 2 * g + 1
            wait_gather(rows0, sem_g0)
            s0 = pltpu.async_copy(rows0, acc_sh.at[dblk.at[j0]], sem_s0, add=True)
            wait_gather(rows1, sem_g1)
            s1 = pltpu.async_copy(rows1, acc_sh.at[dblk.at[j1]], sem_s1, add=True)
            s0.wait()

            @pl.when(j0 + 2 < IB)
            def _():
                pltpu.async_copy(table_hbm.at[sblk.at[j0 + 2]], rows0, sem_g0)
            s1.wait()

            @pl.when(j1 + 2 < IB)
            def _():
                pltpu.async_copy(table_hbm.at[sblk.at[j1 + 2]], rows1, sem_g1)
            return 0
        lax.fori_loop(0, IB // 2, pair, 0)

    plsc.subcore_barrier()

    @pl.when(c == 0)
    def _():
        pltpu.sync_copy(acc_sh.at[pl.ds(row0, ROWS_PER_TILE)],
                        out0_hbm.at[pl.ds(row0, ROWS_PER_TILE)])

    @pl.when(c == 1)
    def _():
        pltpu.sync_copy(acc_sh.at[pl.ds(row0, ROWS_PER_TILE)],
                        out1_hbm.at[pl.ds(row0, ROWS_PER_TILE)])


_sc_spmm = functools.partial(
    pl.kernel,
    out_type=(
        jax.ShapeDtypeStruct((NACC, D), jnp.float32),
        jax.ShapeDtypeStruct((NACC, D), jnp.float32),
    ),
    mesh=plsc.VectorSubcoreMesh(core_axis_name="c", subcore_axis_name="s"),
    scratch_types=[
        pltpu.VMEM((IB, CHUNK), jnp.int32),
        pltpu.VMEM((IB, CHUNK), jnp.int32),
        pltpu.VMEM((IB, CHUNK), jnp.int32),
        pltpu.VMEM((IB, CHUNK), jnp.int32),
        pltpu.VMEM((CHUNK, D), jnp.float32),
        pltpu.VMEM((CHUNK, D), jnp.float32),
        pltpu.VMEM_SHARED((NACC, D), jnp.float32),
        pltpu.SemaphoreType.DMA,
        pltpu.SemaphoreType.DMA,
        pltpu.SemaphoreType.DMA,
        pltpu.SemaphoreType.DMA,
        pltpu.SemaphoreType.DMA,
        pltpu.SemaphoreType.DMA,
    ],
)(_sc_body)


def _combine_body(p0_ref, p1_ref, b_ref, o_ref):
    bias_sum = jnp.sum(b_ref[...], axis=0, keepdims=True)
    o_ref[...] = p0_ref[...] + p1_ref[...] + bias_sum


def _combine(p0, p1, bias):
    BN = 400
    return pl.pallas_call(
        _combine_body,
        grid=(N // BN,),
        in_specs=[
            pl.BlockSpec((BN, D), lambda i: (i, 0)),
            pl.BlockSpec((BN, D), lambda i: (i, 0)),
            pl.BlockSpec((R, D), lambda i: (0, 0)),
        ],
        out_specs=pl.BlockSpec((BN, D), lambda i: (i, 0)),
        out_shape=jax.ShapeDtypeStruct((N, D), jnp.float32),
    )(p0, p1, bias)


@jax.jit
def kernel(inp, edge_index, weights, bias):
    table = _relation_matmuls(inp, weights).reshape(R * N, D)

    roff = (jnp.arange(R, dtype=jnp.int32) * N)[:, None]
    src = (edge_index[:, 1, :] + roff).reshape(-1)
    dst = edge_index[:, 0, :].reshape(-1)
    npad = EP - R * E
    pad_i = jnp.arange(npad, dtype=jnp.int32)
    src = jnp.concatenate([src, pad_i % (R * N)])
    dst = jnp.concatenate([dst, N + pad_i % (NACC - N)])
    src = src.reshape(EP // CHUNK, CHUNK)
    dst = dst.reshape(EP // CHUNK, CHUNK)

    p0, p1 = _sc_spmm(table, src, dst)
    return _combine(p0, p1, bias)
